# speedup vs baseline: 8.0277x; 8.0277x over previous
import functools

import jax
import jax.numpy as jnp
from jax import lax
from jax.experimental import pallas as pl
from jax.experimental.pallas import tpu as pltpu
from jax.experimental.pallas import tpu_sc as plsc

D_MODEL = 128
N_TOKENS = 1024 * 200

NUM_CORES = 2
NUM_SUBCORES = 16
NUM_WORKERS = NUM_CORES * NUM_SUBCORES

B_PER_W = N_TOKENS // NUM_WORKERS
CHUNK = 128
N_CHUNKS = B_PER_W // CHUNK
NBUF = 5
N_GROUPS = N_CHUNKS // NBUF


def _body(tok_hbm, emb_hbm, out_hbm, idx_v, bufs, gsems, osems):
    wid = lax.axis_index("s") * NUM_CORES + lax.axis_index("c")
    base = wid * B_PER_W

    pltpu.sync_copy(tok_hbm.at[wid], idx_v)

    def gather_start(j, b):
        pltpu.async_copy(emb_hbm.at[idx_v.at[j]], bufs[b], gsems[b])

    for b in range(NBUF):
        gather_start(b, b)

    def group(g, _):
        for b in range(NBUF):
            j = g * NBUF + b
            pltpu.make_async_copy(emb_hbm.at[idx_v.at[j]], bufs[b], gsems[b]).wait()
            pltpu.async_copy(
                bufs[b], out_hbm.at[pl.ds(base + j * CHUNK, CHUNK)], osems[b]
            )

            @pl.when(g < N_GROUPS - 1)
            def _():
                pltpu.make_async_copy(
                    bufs[b], out_hbm.at[pl.ds(base + j * CHUNK, CHUNK)], osems[b]
                ).wait()
                gather_start(j + NBUF, b)

        return 0

    lax.fori_loop(0, N_GROUPS, group, 0)

    last = (N_GROUPS - 1) * NBUF
    for b in range(NBUF):
        pltpu.make_async_copy(
            bufs[b], out_hbm.at[pl.ds(base + (last + b) * CHUNK, CHUNK)], osems[b]
        ).wait()


@jax.jit
def _embed(tokens_flat, emb):
    mesh = plsc.VectorSubcoreMesh(core_axis_name="c", subcore_axis_name="s")
    tok3 = tokens_flat.reshape(NUM_WORKERS, N_CHUNKS, CHUNK)
    run = pl.kernel(
        _body,
        out_type=jax.ShapeDtypeStruct((N_TOKENS, D_MODEL), jnp.float32),
        mesh=mesh,
        scratch_types=[
            pltpu.VMEM((N_CHUNKS, CHUNK), jnp.int32),
            [pltpu.VMEM((CHUNK, D_MODEL), jnp.float32) for _ in range(NBUF)],
            [pltpu.SemaphoreType.DMA for _ in range(NBUF)],
            [pltpu.SemaphoreType.DMA for _ in range(NBUF)],
        ],
    )
    return run(tok3, emb)


def kernel(tokens, emb):
    tokens_flat = tokens.reshape(-1).astype(jnp.int32)
    out = _embed(tokens_flat, emb)
    return out.reshape(tokens.shape + (D_MODEL,))
